# baseline (speedup 1.0000x reference)
import functools

import jax
import jax.numpy as jnp
from jax import lax
from jax.experimental import pallas as pl
from jax.experimental.pallas import tpu as pltpu
from jax.experimental.pallas import tpu_sc as plsc

NC = 2
NS = 16
NW = NC * NS
L = 16

B = 16384
D = 32
N_ROWS = 1000000
ROWS_PER_TILE = 128 // D
PACKED_ROWS = N_ROWS // ROWS_PER_TILE

S = 256
NBLK = (N_ROWS // S // 2) * 2
TAIL = N_ROWS - NBLK * S
KMAX = -(-NBLK // NW)

BPW = B // NW
CHUNK = 128
NCHUNK = BPW // CHUNK
GPC = CHUNK // L


def _transpose_body(ut, it, ur, ir, ub2, ib2, ou2, oi2, tb_u, tb_i, tou, toi,
                    sems):
    wid = lax.axis_index("s") * NC + lax.axis_index("c")
    iota = lax.iota(jnp.int32, L)
    OPB = S // ROWS_PER_TILE

    lane_base = (iota & (ROWS_PER_TILE - 1)) * D
    orow_base = iota >> 2

    def emit_rows(n_rows, src_u, src_i, dst_u, dst_i):
        for g in range(n_rows // ROWS_PER_TILE):
            col0 = g * L
            orow_vec = orow_base + g * ROWS_PER_TILE
            for d in range(D):
                lane_vec = lane_base + d
                xu = src_u[d, pl.ds(col0, L)]
                xi = src_i[d, pl.ds(col0, L)]
                plsc.store_scatter(dst_u, [orow_vec, lane_vec], xu)
                plsc.store_scatter(dst_i, [orow_vec, lane_vec], xi)

    def in_copies(k, buf):
        c = (wid + k * NW) * S
        return (pltpu.make_async_copy(ut.at[:, pl.ds(c, S)], ub2.at[buf],
                                      sems.at[buf]),
                pltpu.make_async_copy(it.at[:, pl.ds(c, S)], ib2.at[buf],
                                      sems.at[2 + buf]))

    def out_copies(k, buf):
        orow0 = (wid + k * NW) * OPB
        return (pltpu.make_async_copy(ou2.at[buf], ur.at[pl.ds(orow0, OPB)],
                                      sems.at[4 + buf]),
                pltpu.make_async_copy(oi2.at[buf], ir.at[pl.ds(orow0, OPB)],
                                      sems.at[6 + buf]))

    def valid(k):
        return wid + k * NW < NBLK

    def fire_in(k, buf):
        @pl.when(valid(k))
        def _():
            for cp in in_copies(k, buf):
                cp.start()

    for par in (0, 1):
        fire_in(par, par)

    def step(t, carry):
        for par in (0, 1):
            k = 2 * t + par
            buf = par

            @pl.when(valid(k))
            def _(k=k, buf=buf):
                for cp in in_copies(k, buf):
                    cp.wait()

            @pl.when((k >= 2) & valid(k - 2))
            def _(k=k, buf=buf):
                for cp in out_copies(k - 2, buf):
                    cp.wait()

            @pl.when(valid(k))
            def _(k=k, buf=buf):
                emit_rows(OPB, ub2.at[buf], ib2.at[buf],
                          ou2.at[buf], oi2.at[buf])
                for cp in out_copies(k, buf):
                    cp.start()

            fire_in(k + 2, buf)
        return carry

    lax.fori_loop(0, (KMAX + 3) // 2 + 1, step, None)

    @pl.when(wid == NW - 1)
    def _():
        c = pl.multiple_of((wid - (NW - 1)) + NBLK * S, 128)
        cu = pltpu.async_copy(ut.at[:, pl.ds(c, 128)], tb_u, sems.at[0])
        ci = pltpu.async_copy(it.at[:, pl.ds(c, 128)], tb_i, sems.at[1])
        cu.wait()
        ci.wait()
        n = TAIL // ROWS_PER_TILE
        emit_rows(n, tb_u, tb_i, tou, toi)
        orow0 = PACKED_ROWS - n
        co = pltpu.async_copy(tou, ur.at[pl.ds(orow0, n)], sems.at[2])
        co2 = pltpu.async_copy(toi, ir.at[pl.ds(orow0, n)], sems.at[3])
        co.wait()
        co2.wait()


def _gather_body(uid_hbm, iid_hbm, ut_hbm, it_hbm, out_hbm,
                 idx_u, idx_i, hi_u, hi_i, u_bufs, i_bufs, out_v, sems):
    wid = lax.axis_index("s") * NC + lax.axis_index("c")
    base = wid * BPW

    for c in range(NCHUNK):
        pltpu.sync_copy(uid_hbm.at[wid * NCHUNK + c], idx_u.at[c])
        pltpu.sync_copy(iid_hbm.at[wid * NCHUNK + c], idx_i.at[c])

    def split(c):
        for k in range(GPC):
            sl = pl.ds(k * L, L)
            hi_u[c, sl] = idx_u[c, sl] >> 2
            hi_i[c, sl] = idx_i[c, sl] >> 2

    def fire(c):
        buf = c % 2
        cu = pltpu.async_copy(ut_hbm.at[hi_u.at[c]], u_bufs.at[buf],
                              sems.at[buf])
        ci = pltpu.async_copy(it_hbm.at[hi_i.at[c]], i_bufs.at[buf],
                              sems.at[2 + buf])
        return cu, ci

    split(0)
    inflight = {0: fire(0)}
    split(1)
    inflight[1] = fire(1)
    split(2)
    split(3)

    iota = lax.iota(jnp.int32, L)

    for c in range(NCHUNK):
        buf = c % 2
        cu, ci = inflight.pop(c)
        cu.wait()
        ci.wait()

        def group(g, carry, c=c, buf=buf):
            sl = pl.ds(g * L, L)
            row = g * L + iota
            lo_u = (idx_u[c, sl] & (ROWS_PER_TILE - 1)) * D
            lo_i = (idx_i[c, sl] & (ROWS_PER_TILE - 1)) * D
            acc = jnp.zeros((L,), jnp.float32)
            for d in range(D):
                u = plsc.load_gather(u_bufs.at[buf], [row, lo_u + d])
                v = plsc.load_gather(i_bufs.at[buf], [row, lo_i + d])
                acc = acc + u * v
            out_v[pl.ds(c * CHUNK + g * L, L)] = acc
            return carry

        lax.fori_loop(0, GPC, group, None)
        if c + 2 < NCHUNK:
            inflight[c + 2] = fire(c + 2)

    pltpu.sync_copy(out_v, out_hbm.at[pl.ds(base, BPW)])


def _mesh():
    return plsc.VectorSubcoreMesh(core_axis_name="c", subcore_axis_name="s",
                                  num_cores=NC, num_subcores=NS)


@functools.cache
def _build_transpose():
    return pl.kernel(
        _transpose_body,
        out_type=(jax.ShapeDtypeStruct((PACKED_ROWS, 128), jnp.float32),
                  jax.ShapeDtypeStruct((PACKED_ROWS, 128), jnp.float32)),
        mesh=_mesh(),
        compiler_params=pltpu.CompilerParams(needs_layout_passes=False,
                                             disable_bounds_checks=True),
        scratch_types=[
            pltpu.VMEM((2, D, S), jnp.float32),
            pltpu.VMEM((2, D, S), jnp.float32),
            pltpu.VMEM((2, S // ROWS_PER_TILE, 128), jnp.float32),
            pltpu.VMEM((2, S // ROWS_PER_TILE, 128), jnp.float32),
            pltpu.VMEM((D, 128), jnp.float32),
            pltpu.VMEM((D, 128), jnp.float32),
            pltpu.VMEM((TAIL // ROWS_PER_TILE, 128), jnp.float32),
            pltpu.VMEM((TAIL // ROWS_PER_TILE, 128), jnp.float32),
            pltpu.SemaphoreType.DMA((8,)),
        ],
    )


@functools.cache
def _build_gather():
    return pl.kernel(
        _gather_body,
        out_type=jax.ShapeDtypeStruct((B,), jnp.float32),
        mesh=_mesh(),
        compiler_params=pltpu.CompilerParams(needs_layout_passes=False),
        scratch_types=[
            pltpu.VMEM((NCHUNK, CHUNK), jnp.int32),
            pltpu.VMEM((NCHUNK, CHUNK), jnp.int32),
            pltpu.VMEM((NCHUNK, CHUNK), jnp.int32),
            pltpu.VMEM((NCHUNK, CHUNK), jnp.int32),
            pltpu.VMEM((2, CHUNK, 128), jnp.float32),
            pltpu.VMEM((2, CHUNK, 128), jnp.float32),
            pltpu.VMEM((BPW,), jnp.float32),
            pltpu.SemaphoreType.DMA((4,)),
        ],
    )


@jax.jit
def kernel(user_id, item_id, user_table, item_table):
    uid = user_id.astype(jnp.int32).reshape(NW * NCHUNK, CHUNK)
    iid = item_id.astype(jnp.int32).reshape(NW * NCHUNK, CHUNK)
    ur, ir = _build_transpose()(user_table.T, item_table.T)
    return _build_gather()(uid, iid, ur, ir)


# speedup vs baseline: 1.3972x; 1.3972x over previous
import functools

import jax
import jax.numpy as jnp
from jax import lax
from jax.experimental import pallas as pl
from jax.experimental.pallas import tpu as pltpu
from jax.experimental.pallas import tpu_sc as plsc

NC = 2
NS = 16
NW = NC * NS
L = 16

B = 16384
D = 32
BPW = B // NW
CHUNK = 128
NCHUNK = BPW // CHUNK
GROUPS = BPW // L


def _body(uid_hbm, iid_hbm, ut_hbm, it_hbm, out_hbm,
          idx_u, idx_i, u_rows, i_rows, out_v, sem_u, sem_i):
    wid = lax.axis_index("s") * NC + lax.axis_index("c")
    base = wid * BPW

    pltpu.sync_copy(uid_hbm.at[wid], idx_u)
    pltpu.sync_copy(iid_hbm.at[wid], idx_i)

    copies = []
    for j in range(NCHUNK):
        copies.append(pltpu.async_copy(
            ut_hbm.at[idx_u.at[j]], u_rows.at[pl.ds(j * CHUNK, CHUNK)], sem_u))
        copies.append(pltpu.async_copy(
            it_hbm.at[idx_i.at[j]], i_rows.at[pl.ds(j * CHUNK, CHUNK)], sem_i))
    for cp in copies:
        cp.wait()

    iota = lax.iota(jnp.int32, L)

    def group(g, carry):
        base_row = g * L
        vals = jnp.zeros((L,), jnp.float32)
        for r in range(L):
            row = base_row + r
            prod = (u_rows[row, pl.ds(0, L)] * i_rows[row, pl.ds(0, L)]
                    + u_rows[row, pl.ds(L, L)] * i_rows[row, pl.ds(L, L)])
            vals = jnp.where(iota == r, jnp.sum(prod), vals)
        out_v[pl.ds(base_row, L)] = vals
        return carry

    lax.fori_loop(0, GROUPS, group, None)

    pltpu.sync_copy(out_v, out_hbm.at[pl.ds(base, BPW)])


@functools.cache
def _build():
    return pl.kernel(
        _body,
        out_type=jax.ShapeDtypeStruct((B,), jnp.float32),
        mesh=plsc.VectorSubcoreMesh(core_axis_name="c", subcore_axis_name="s",
                                    num_cores=NC, num_subcores=NS),
        compiler_params=pltpu.CompilerParams(needs_layout_passes=False,
                                             use_tc_tiling_on_sc=False),
        scratch_types=[
            pltpu.VMEM((NCHUNK, CHUNK), jnp.int32),
            pltpu.VMEM((NCHUNK, CHUNK), jnp.int32),
            pltpu.VMEM((BPW, D), jnp.float32),
            pltpu.VMEM((BPW, D), jnp.float32),
            pltpu.VMEM((BPW,), jnp.float32),
            pltpu.SemaphoreType.DMA,
            pltpu.SemaphoreType.DMA,
        ],
    )


@jax.jit
def kernel(user_id, item_id, user_table, item_table):
    uid = user_id.astype(jnp.int32).reshape(NW, NCHUNK, CHUNK)
    iid = item_id.astype(jnp.int32).reshape(NW, NCHUNK, CHUNK)
    return _build()(uid, iid, user_table, item_table)
